# baseline (speedup 1.0000x reference)
import functools

import jax
import jax.numpy as jnp
from jax import lax
from jax.experimental import pallas as pl
from jax.experimental.pallas import tpu as pltpu
from jax.experimental.pallas import tpu_sc as plsc

VOCAB = 1000000
EMB = 64
BATCH = 4096

_NC, _NS = 2, 16
_NW = _NC * _NS
_B_PER_W = BATCH // _NW


@functools.cache
def _make_sc_gather():
    mesh = plsc.VectorSubcoreMesh(core_axis_name="c", subcore_axis_name="s")

    @functools.partial(
        pl.kernel,
        mesh=mesh,
        out_type=[
            jax.ShapeDtypeStruct((BATCH, EMB), jnp.float32),
            jax.ShapeDtypeStruct((BATCH, EMB), jnp.float32),
        ],
        scratch_types=[
            pltpu.VMEM((_B_PER_W,), jnp.int32),
            pltpu.SemaphoreType.DMA,
        ],
        compiler_params=pltpu.CompilerParams(needs_layout_passes=False),
    )
    def _sc_gather(center_hbm, context_hbm, table_hbm, out_c_hbm, out_x_hbm,
                   idx_v, sem):
        wid = lax.axis_index("s") * _NC + lax.axis_index("c")
        base = wid * _B_PER_W

        for words_hbm, out_hbm in ((center_hbm, out_c_hbm),
                                   (context_hbm, out_x_hbm)):
            pltpu.sync_copy(words_hbm.at[pl.ds(base, _B_PER_W)], idx_v)

            for g in range(_B_PER_W // 16):
                v = idx_v[pl.ds(16 * g, 16)]
                for i in range(16):
                    w = 16 * g + i
                    pltpu.async_copy(
                        table_hbm.at[v[i]], out_hbm.at[base + w], sem)
            pltpu.make_async_copy(
                table_hbm.at[pl.ds(0, _B_PER_W)],
                out_hbm.at[pl.ds(base, _B_PER_W)], sem).wait()

    return _sc_gather


_BM = 512


def _mm_body(a_ref, b_ref, o_ref):
    o_ref[...] = lax.dot_general(
        a_ref[...].astype(jnp.bfloat16), b_ref[...].astype(jnp.bfloat16),
        (((1,), (1,)), ((), ())),
        preferred_element_type=jnp.float32,
    )


def _matmul(center_emb, context_emb):
    return pl.pallas_call(
        _mm_body,
        grid=(BATCH // _BM,),
        in_specs=[
            pl.BlockSpec((_BM, EMB), lambda i: (i, 0)),
            pl.BlockSpec((BATCH, EMB), lambda i: (0, 0)),
        ],
        out_specs=pl.BlockSpec((_BM, BATCH), lambda i: (i, 0)),
        out_shape=jax.ShapeDtypeStruct((BATCH, BATCH), jnp.float32),
        compiler_params=pltpu.CompilerParams(
            dimension_semantics=("arbitrary",),
        ),
    )(center_emb, context_emb)


def kernel(center_words, context_words, embeddings):
    center_emb, context_emb = _make_sc_gather()(
        center_words, context_words, embeddings)
    return _matmul(center_emb, context_emb)


# speedup vs baseline: 1.3343x; 1.3343x over previous
import functools

import jax
import jax.numpy as jnp
from jax import lax
from jax.experimental import pallas as pl
from jax.experimental.pallas import tpu as pltpu
from jax.experimental.pallas import tpu_sc as plsc

VOCAB = 1000000
EMB = 64
BATCH = 4096

_NC, _NS = 2, 16
_NW = _NC * _NS
_B_PER_W = BATCH // _NW


@functools.cache
def _make_sc_gather():
    mesh = plsc.VectorSubcoreMesh(core_axis_name="c", subcore_axis_name="s")

    @functools.partial(
        pl.kernel,
        mesh=mesh,
        out_type=[
            jax.ShapeDtypeStruct((BATCH, EMB), jnp.float32),
            jax.ShapeDtypeStruct((BATCH, EMB), jnp.float32),
        ],
        scratch_types=[
            pltpu.VMEM((_B_PER_W,), jnp.int32),
            pltpu.VMEM((_B_PER_W, EMB), jnp.float32),
            pltpu.SemaphoreType.DMA,
        ],
        compiler_params=pltpu.CompilerParams(needs_layout_passes=False),
    )
    def _sc_gather(center_hbm, context_hbm, table_hbm, out_c_hbm, out_x_hbm,
                   idx_v, out_rows_v, sem):
        wid = lax.axis_index("s") * _NC + lax.axis_index("c")
        base = wid * _B_PER_W

        for words_hbm, out_hbm in ((center_hbm, out_c_hbm),
                                   (context_hbm, out_x_hbm)):
            pltpu.sync_copy(words_hbm.at[pl.ds(base, _B_PER_W)], idx_v)

            for g in range(_B_PER_W // 16):
                v = idx_v[pl.ds(16 * g, 16)]
                for i in range(16):
                    w = 16 * g + i
                    pltpu.async_copy(
                        table_hbm.at[v[i]], out_rows_v.at[w], sem)
            pltpu.make_async_copy(
                table_hbm.at[pl.ds(0, _B_PER_W)], out_rows_v, sem).wait()
            pltpu.sync_copy(out_rows_v, out_hbm.at[pl.ds(base, _B_PER_W)])

    return _sc_gather


_BM = 512


def _mm_body(a_ref, b_ref, o_ref):
    o_ref[...] = lax.dot_general(
        a_ref[...].astype(jnp.bfloat16), b_ref[...].astype(jnp.bfloat16),
        (((1,), (1,)), ((), ())),
        preferred_element_type=jnp.float32,
    )


def _matmul(center_emb, context_emb):
    return pl.pallas_call(
        _mm_body,
        grid=(BATCH // _BM,),
        in_specs=[
            pl.BlockSpec((_BM, EMB), lambda i: (i, 0)),
            pl.BlockSpec((BATCH, EMB), lambda i: (0, 0)),
        ],
        out_specs=pl.BlockSpec((_BM, BATCH), lambda i: (i, 0)),
        out_shape=jax.ShapeDtypeStruct((BATCH, BATCH), jnp.float32),
        compiler_params=pltpu.CompilerParams(
            dimension_semantics=("arbitrary",),
        ),
    )(center_emb, context_emb)


def kernel(center_words, context_words, embeddings):
    center_emb, context_emb = _make_sc_gather()(
        center_words, context_words, embeddings)
    return _matmul(center_emb, context_emb)


# speedup vs baseline: 1.3344x; 1.0000x over previous
import functools

import jax
import jax.numpy as jnp
from jax import lax
from jax.experimental import pallas as pl
from jax.experimental.pallas import tpu as pltpu
from jax.experimental.pallas import tpu_sc as plsc

VOCAB = 1000000
EMB = 64
BATCH = 4096

_NC, _NS = 2, 16
_NW = _NC * _NS
_B_PER_W = BATCH // _NW


@functools.cache
def _make_sc_gather():
    mesh = plsc.VectorSubcoreMesh(core_axis_name="c", subcore_axis_name="s")

    @functools.partial(
        pl.kernel,
        mesh=mesh,
        out_type=[
            jax.ShapeDtypeStruct((BATCH, EMB), jnp.float32),
            jax.ShapeDtypeStruct((BATCH, EMB), jnp.float32),
        ],
        scratch_types=[
            pltpu.VMEM((_B_PER_W,), jnp.int32),
            pltpu.VMEM((_B_PER_W, EMB), jnp.float32),
            [pltpu.SemaphoreType.DMA] * 8,
        ],
        compiler_params=pltpu.CompilerParams(needs_layout_passes=False),
    )
    def _sc_gather(center_hbm, context_hbm, table_hbm, out_c_hbm, out_x_hbm,
                   idx_v, out_rows_v, sems):
        wid = lax.axis_index("s") * _NC + lax.axis_index("c")
        base = wid * _B_PER_W

        for words_hbm, out_hbm in ((center_hbm, out_c_hbm),
                                   (context_hbm, out_x_hbm)):
            pltpu.sync_copy(words_hbm.at[pl.ds(base, _B_PER_W)], idx_v)

            for g in range(_B_PER_W // 16):
                v = idx_v[pl.ds(16 * g, 16)]
                for i in range(16):
                    w = 16 * g + i
                    pltpu.async_copy(
                        table_hbm.at[v[i]], out_rows_v.at[w], sems[w % 8])
            for q in range(8):
                pltpu.make_async_copy(
                    table_hbm.at[pl.ds(0, _B_PER_W // 8)],
                    out_rows_v.at[pl.ds(0, _B_PER_W // 8)], sems[q]).wait()
            pltpu.sync_copy(out_rows_v, out_hbm.at[pl.ds(base, _B_PER_W)])

    return _sc_gather


_BM = 512


def _mm_body(a_ref, b_ref, o_ref):
    o_ref[...] = lax.dot_general(
        a_ref[...].astype(jnp.bfloat16), b_ref[...].astype(jnp.bfloat16),
        (((1,), (1,)), ((), ())),
        preferred_element_type=jnp.float32,
    )


def _matmul(center_emb, context_emb):
    return pl.pallas_call(
        _mm_body,
        grid=(BATCH // _BM,),
        in_specs=[
            pl.BlockSpec((_BM, EMB), lambda i: (i, 0)),
            pl.BlockSpec((BATCH, EMB), lambda i: (0, 0)),
        ],
        out_specs=pl.BlockSpec((_BM, BATCH), lambda i: (i, 0)),
        out_shape=jax.ShapeDtypeStruct((BATCH, BATCH), jnp.float32),
        compiler_params=pltpu.CompilerParams(
            dimension_semantics=("arbitrary",),
        ),
    )(center_emb, context_emb)


def kernel(center_words, context_words, embeddings):
    center_emb, context_emb = _make_sc_gather()(
        center_words, context_words, embeddings)
    return _matmul(center_emb, context_emb)
